# baseline (speedup 1.0000x reference)
import functools

import jax
import jax.numpy as jnp
from jax import lax
from jax.experimental import pallas as pl
from jax.experimental.pallas import tpu as pltpu
from jax.experimental.pallas import tpu_sc as plsc

B = 16384
V = 1000000
D = 32
F_ITEM = 1065
H_ITEM = 200
F_USER = 4
NC = 2
NS = 16
NW = NC * NS
BPW = B // NW
BT = 512
NB = B // BT


def _gather_table(ids_v, tab_lin, idxmat, gbuf, out_h, base, sem):
    for blk in range(BPW // 16):
        idsv = ids_v[pl.ds(blk * 16, 16)]
        for j in range(D):
            idxmat[j, pl.ds(blk * 16, 16)] = idsv + j * V
    copies = [
        pltpu.async_copy(tab_lin.at[idxmat.at[j]], gbuf.at[j], sem)
        for j in range(D)
    ]
    for c in copies:
        c.wait()
    for j in range(D):
        pltpu.sync_copy(gbuf.at[j], out_h.at[j, pl.ds(base, BPW)])


def _sc_gather(uid, iid, uW_lin, uBf, iW_lin, iBf):
    mesh = plsc.VectorSubcoreMesh(core_axis_name="c", subcore_axis_name="s")

    @functools.partial(
        pl.kernel,
        mesh=mesh,
        compiler_params=pltpu.CompilerParams(use_tc_tiling_on_sc=False),
        out_type=[
            jax.ShapeDtypeStruct((D, B), jnp.float32),
            jax.ShapeDtypeStruct((D, B), jnp.float32),
            jax.ShapeDtypeStruct((B,), jnp.float32),
            jax.ShapeDtypeStruct((B,), jnp.float32),
        ],
        scratch_types=[
            pltpu.VMEM((BPW,), jnp.int32),
            pltpu.VMEM((BPW,), jnp.int32),
            pltpu.VMEM((D, BPW), jnp.int32),
            pltpu.VMEM((D, BPW), jnp.float32),
            pltpu.VMEM((BPW,), jnp.float32),
            pltpu.VMEM((BPW,), jnp.float32),
            pltpu.SemaphoreType.DMA,
            pltpu.SemaphoreType.DMA,
        ],
    )
    def k(uid_h, iid_h, uW_h, uBf_h, iW_h, iBf_h, gut_h, git_h,
          gub_h, gib_h, ids_u_v, ids_i_v, idxmat, gbuf, bu_v, bi_v,
          semw, semb):
        wid = lax.axis_index("s") * NC + lax.axis_index("c")
        base = wid * BPW
        pltpu.sync_copy(uid_h.at[pl.ds(base, BPW)], ids_u_v)
        pltpu.sync_copy(iid_h.at[pl.ds(base, BPW)], ids_i_v)
        cbu = pltpu.async_copy(uBf_h.at[ids_u_v], bu_v, semb)
        cbi = pltpu.async_copy(iBf_h.at[ids_i_v], bi_v, semb)
        _gather_table(ids_u_v, uW_h, idxmat, gbuf, gut_h, base, semw)
        _gather_table(ids_i_v, iW_h, idxmat, gbuf, git_h, base, semw)
        cbu.wait()
        cbi.wait()
        pltpu.sync_copy(bu_v, gub_h.at[pl.ds(base, BPW)])
        pltpu.sync_copy(bi_v, gib_h.at[pl.ds(base, BPW)])

    return k(uid, iid, uW_lin, uBf, iW_lin, iBf)


def _tc_body(feats_t, w1t, b1, w2t, b2, uft, wut, bu1, gut_r, git_r,
             gub_r, gib_r, out):
    h = jnp.maximum(
        jnp.dot(w1t[:], feats_t[:], preferred_element_type=jnp.float32)
        + b1[:], 0.0)
    im = jnp.dot(w2t[:], h, preferred_element_type=jnp.float32) + b2[:]
    um = jnp.dot(wut[:], uft[:], preferred_element_type=jnp.float32) + bu1[:]
    ue = gut_r[:] + um
    ie = git_r[:] + im
    out[:] = jnp.sum(ue * ie, axis=0) + gub_r[:] + gib_r[:]


def _tc_compute(feats_t, w1t, b1, w2t, b2, uft, wut, bu1, gut, git, gub, gib):
    return pl.pallas_call(
        _tc_body,
        grid=(NB,),
        in_specs=[
            pl.BlockSpec((F_ITEM, BT), lambda i: (0, i)),
            pl.BlockSpec((H_ITEM, F_ITEM), lambda i: (0, 0)),
            pl.BlockSpec((H_ITEM, 1), lambda i: (0, 0)),
            pl.BlockSpec((D, H_ITEM), lambda i: (0, 0)),
            pl.BlockSpec((D, 1), lambda i: (0, 0)),
            pl.BlockSpec((F_USER, BT), lambda i: (0, i)),
            pl.BlockSpec((D, F_USER), lambda i: (0, 0)),
            pl.BlockSpec((D, 1), lambda i: (0, 0)),
            pl.BlockSpec((D, BT), lambda i: (0, i)),
            pl.BlockSpec((D, BT), lambda i: (0, i)),
            pl.BlockSpec((BT,), lambda i: (i,)),
            pl.BlockSpec((BT,), lambda i: (i,)),
        ],
        out_specs=pl.BlockSpec((BT,), lambda i: (i,)),
        out_shape=jax.ShapeDtypeStruct((B,), jnp.float32),
        compiler_params=pltpu.CompilerParams(
            dimension_semantics=("arbitrary",)),
    )(feats_t, w1t, b1, w2t, b2, uft, wut, bu1, gut, git, gub, gib)


def kernel(user_id, user_feats, item_id, item_feats, ids_active_users,
           ids_active_items, user_W, user_B, item_W, item_B,
           w_u1, b_u1, w_i1, b_i1, w_i2, b_i2):
    uid = user_id.astype(jnp.int32)
    iid = item_id.astype(jnp.int32)
    gut, git, gub, gib = _sc_gather(uid, iid, user_W.T.reshape(D * V),
                                    user_B.reshape(V),
                                    item_W.T.reshape(D * V),
                                    item_B.reshape(V))
    return _tc_compute(item_feats.T, w_i1.T, b_i1.reshape(-1, 1), w_i2.T,
                       b_i2.reshape(-1, 1), user_feats.T, w_u1.T,
                       b_u1.reshape(-1, 1), gut, git, gub, gib)
